# baseline (speedup 1.0000x reference)
import functools

import jax
import jax.numpy as jnp
from jax import lax
from jax.experimental import pallas as pl
from jax.experimental.pallas import tpu as pltpu
from jax.experimental.pallas import tpu_sc as plsc

N = 2097152
NW = 32
PER_W = N // NW
C = 2048
NCH = PER_W // C
NB = C // 128
NVPB = 128 // 16


def _axis_parts(v):
    ix = v * 128.0 + 127.5
    i0 = (ix + 256.0).astype(jnp.int32) - 256
    w = ix - i0.astype(jnp.float32)
    neg = i0 < 0
    hi = i0 >= 255
    c = jnp.minimum(jnp.maximum(i0, 0), 255)
    a = jnp.where(neg, w, 1.0 - w)
    b = jnp.where(neg | hi, 0.0, w)
    return c, a, b


def _body(xs, ys, zs, tbl, out, xb, yb, zb, axb, bxb, ayb, byb, azb, bzb,
          s8b, idxb, gbuf, obuf, sem):
    wid = lax.axis_index("s") * 2 + lax.axis_index("c")
    base = wid * PER_W

    def chunk(j, carry):
        off = base + j * C

        def comp_row(r, c2):
            for u in range(NVPB):
                sl = pl.ds(r * 128 + u * 16, 16)
                xc, ax, bx = _axis_parts(xb[sl])
                yc, ay, by = _axis_parts(yb[sl])
                zc, az, bz = _axis_parts(zb[sl])
                flat = zc * 65536 + yc * 256 + xc
                idxb[r, pl.ds(u * 16, 16)] = lax.shift_right_logical(flat, 2)
                s8b[sl] = lax.shift_left(flat & 3, 3)
                axb[sl] = ax
                bxb[sl] = bx
                ayb[sl] = ay
                byb[sl] = by
                azb[sl] = az
                bzb[sl] = bz
            return c2


        def ext_row(r, c2):
            for u in range(NVPB):
                sl = pl.ds(r * 128 + u * 16, 16)
                w = gbuf[r, pl.ds(u * 16, 16)]
                wsh = lax.shift_right_logical(w, s8b[sl])
                ax = axb[sl]
                bx = bxb[sl]

                def dot(dz, dy):
                    sh = dz * 4 + dy * 2
                    t = lax.shift_right_logical(wsh, sh) if sh else wsh
                    b0 = (t & 1).astype(jnp.float32)
                    b1 = (lax.shift_right_logical(t, 1) & 1).astype(jnp.float32)
                    return ax * b0 + bx * b1

                sz0 = ayb[sl] * dot(0, 0) + byb[sl] * dot(0, 1)
                sz1 = ayb[sl] * dot(1, 0) + byb[sl] * dot(1, 1)
                obuf[sl] = azb[sl] * sz0 + bzb[sl] * sz1
            return c2

        return carry

    lax.fori_loop(0, NCH, chunk, 0, unroll=False)


_mesh = plsc.VectorSubcoreMesh(core_axis_name="c", subcore_axis_name="s")

_sc_call = functools.partial(
    pl.kernel,
    mesh=_mesh,
    compiler_params=pltpu.CompilerParams(use_tc_tiling_on_sc=True),
    out_type=jax.ShapeDtypeStruct((N,), jnp.float32),
    scratch_types=[
        pltpu.VMEM((C,), jnp.float32),
        pltpu.VMEM((C,), jnp.float32),
        pltpu.VMEM((C,), jnp.float32),
        pltpu.VMEM((C,), jnp.float32),
        pltpu.VMEM((C,), jnp.float32),
        pltpu.VMEM((C,), jnp.float32),
        pltpu.VMEM((C,), jnp.float32),
        pltpu.VMEM((C,), jnp.float32),
        pltpu.VMEM((C,), jnp.float32),
        pltpu.VMEM((C,), jnp.int32),
        pltpu.VMEM((NB, 128), jnp.int32),
        pltpu.VMEM((NB, 128), jnp.int32),
        pltpu.VMEM((C,), jnp.float32),
        pltpu.SemaphoreType.DMA,
    ],
)(_body)


def _pack_table(grid):
    b = grid.astype(jnp.uint8)
    px = b | (jnp.pad(b[:, :, 1:], ((0, 0), (0, 0), (0, 1))) << 1)
    pxy = px | (jnp.pad(px[:, 1:, :], ((0, 0), (0, 1), (0, 0))) << 2)
    pxyz = pxy | (jnp.pad(pxy[1:, :, :], ((0, 1), (0, 0), (0, 0))) << 4)
    p4 = pxyz.reshape(-1, 4).astype(jnp.uint32)
    tbl = p4[:, 0] | (p4[:, 1] << 8) | (p4[:, 2] << 16) | (p4[:, 3] << 24)
    return lax.bitcast_convert_type(tbl, jnp.int32)


def kernel(coords, grid):
    flat = coords.reshape(-1)
    xs = flat[:N]
    ys = flat[:N]
    zs = flat[:N]
    tbl = jnp.zeros((4194304,), jnp.int32)
    return _sc_call(xs, ys, zs, tbl)


# speedup vs baseline: 4.9733x; 4.9733x over previous
import functools

import jax
import jax.numpy as jnp
from jax import lax
from jax.experimental import pallas as pl
from jax.experimental.pallas import tpu as pltpu
from jax.experimental.pallas import tpu_sc as plsc

N = 2097152
NW = 32
PER_W = N // NW
C = 2048
NCH = PER_W // C
NB = C // 128
ROWS_IN = 4 * NB


def _cell(v):
    ix = v * 128.0 + 127.5
    i0 = (ix + 256.0).astype(jnp.int32) - 256
    return i0, jnp.minimum(jnp.maximum(i0, 0), 255)


def _axis_parts(v):
    ix = v * 128.0 + 127.5
    i0 = (ix + 256.0).astype(jnp.int32) - 256
    w = ix - i0.astype(jnp.float32)
    neg = i0 < 0
    a = jnp.where(neg, w, 1.0 - w)
    b = jnp.where(neg | (i0 >= 255), 0.0, w)
    return a, b


def _idx_body(cpl, pk, cbuf, pkb):
    wid = lax.axis_index("s") * 2 + lax.axis_index("c")

    def chunk(j, carry):
        pltpu.sync_copy(cpl.at[pl.ds((wid * NCH + j) * ROWS_IN, ROWS_IN)], cbuf)

        def comp_row(r, c2):
            for u in range(8):
                sl = pl.ds(u * 16, 16)
                _, xc = _cell(cbuf[4 * r + 0, sl])
                iy, yc = _cell(cbuf[4 * r + 1, sl])
                _, zc = _cell(cbuf[4 * r + 2, sl])
                pkb[r, sl] = (zc * 16384
                              + lax.shift_left(lax.shift_right_logical(yc, 2), 8)
                              + xc
                              + lax.shift_left(yc & 3, 22))
            return c2

        lax.fori_loop(0, NB, comp_row, 0, unroll=False)
        pltpu.sync_copy(pkb, pk.at[pl.ds((wid * NCH + j) * NB, NB)])
        return carry

    lax.fori_loop(0, NCH, chunk, 0, unroll=False)


def _gat_body(cpl, pk, tbl, out, cbuf, pkbuf, idxb, gbuf, obuf,
              sem0, sem1, osem0, osem1):
    wid = lax.axis_index("s") * 2 + lax.axis_index("c")
    sems = [sem0, sem1]
    osems = [osem0, osem1]

    def stage(j, slot):
        pltpu.sync_copy(pk.at[pl.ds((wid * NCH + j) * NB, NB)], pkbuf.at[slot])
        pltpu.sync_copy(cpl.at[pl.ds((wid * NCH + j) * ROWS_IN, ROWS_IN)],
                        cbuf.at[slot])

        def mask_row(r, c2):
            for u in range(8):
                sl = pl.ds(u * 16, 16)
                idxb[slot, r, sl] = pkbuf[slot, r, sl] & 0x3FFFFF
            return c2

        lax.fori_loop(0, NB, mask_row, 0, unroll=False)
        for r in range(NB):
            pltpu.async_copy(tbl.at[idxb.at[slot, r]], gbuf.at[slot, r],
                             sems[slot])

    def finish(j, slot):
        for r in range(NB):
            pltpu.make_async_copy(tbl.at[idxb.at[slot, r]],
                                  gbuf.at[slot, r], sems[slot]).wait()

        @pl.when(j >= 2)
        def _():
            pltpu.make_async_copy(
                obuf.at[slot], out.at[pl.ds((wid * NCH + j) * NB, NB)],
                osems[slot]).wait()

        def ext_row(r, c2):
            for u in range(8):
                sl = pl.ds(u * 16, 16)
                ax, bx = _axis_parts(cbuf[slot, 4 * r + 0, sl])
                ay, by = _axis_parts(cbuf[slot, 4 * r + 1, sl])
                az, bz = _axis_parts(cbuf[slot, 4 * r + 2, sl])
                w = gbuf[slot, r, sl]
                s8 = lax.shift_left(
                    lax.shift_right_logical(pkbuf[slot, r, sl], 22), 3)
                wsh = lax.shift_right_logical(w, s8)

                def dot(dz, dy):
                    sh = dz * 4 + dy * 2
                    t = lax.shift_right_logical(wsh, sh) if sh else wsh
                    b0 = (t & 1).astype(jnp.float32)
                    b1 = (lax.shift_right_logical(t, 1) & 1).astype(jnp.float32)
                    return ax * b0 + bx * b1

                sz0 = ay * dot(0, 0) + by * dot(0, 1)
                sz1 = ay * dot(1, 0) + by * dot(1, 1)
                obuf[slot, r, sl] = az * sz0 + bz * sz1
            return c2

        lax.fori_loop(0, NB, ext_row, 0, unroll=False)
        pltpu.async_copy(obuf.at[slot], out.at[pl.ds((wid * NCH + j) * NB, NB)],
                         osems[slot])

    stage(0, 0)

    def pair(p, carry):
        stage(2 * p + 1, 1)
        finish(2 * p, 0)

        @pl.when(p < NCH // 2 - 1)
        def _():
            stage(2 * p + 2, 0)

        finish(2 * p + 1, 1)
        return carry

    lax.fori_loop(0, NCH // 2, pair, 0, unroll=False)
    for slot in (0, 1):
        pltpu.make_async_copy(
            obuf.at[slot], out.at[pl.ds(wid * NCH * NB, NB)],
            osems[slot]).wait()


_mesh = plsc.VectorSubcoreMesh(core_axis_name="c", subcore_axis_name="s")

_idx_call = functools.partial(
    pl.kernel,
    mesh=_mesh,
    out_type=jax.ShapeDtypeStruct((N // 128, 128), jnp.int32),
    scratch_types=[
        pltpu.VMEM((ROWS_IN, 128), jnp.float32),
        pltpu.VMEM((NB, 128), jnp.int32),
    ],
)(_idx_body)

_gat_call = functools.partial(
    pl.kernel,
    mesh=_mesh,
    out_type=jax.ShapeDtypeStruct((N // 128, 128), jnp.float32),
    scratch_types=[
        pltpu.VMEM((2, ROWS_IN, 128), jnp.float32),
        pltpu.VMEM((2, NB, 128), jnp.int32),
        pltpu.VMEM((2, NB, 128), jnp.int32),
        pltpu.VMEM((2, NB, 128), jnp.int32),
        pltpu.VMEM((2, NB, 128), jnp.float32),
        pltpu.SemaphoreType.DMA,
        pltpu.SemaphoreType.DMA,
        pltpu.SemaphoreType.DMA,
        pltpu.SemaphoreType.DMA,
    ],
)(_gat_body)


def _pack_table(grid):
    b = lax.optimization_barrier(grid.astype(jnp.uint8))
    px = b | (jnp.pad(b[:, :, 1:], ((0, 0), (0, 0), (0, 1))) << 1)
    pxy = px | (jnp.pad(px[:, 1:, :], ((0, 0), (0, 1), (0, 0))) << 2)
    pxyz = pxy | (jnp.pad(pxy[1:, :, :], ((0, 1), (0, 0), (0, 0))) << 4)
    v = pxyz.reshape(256, 64, 4, 256)
    w = (v[:, :, 0, :].astype(jnp.uint32)
         | (v[:, :, 1, :].astype(jnp.uint32) << 8)
         | (v[:, :, 2, :].astype(jnp.uint32) << 16)
         | (v[:, :, 3, :].astype(jnp.uint32) << 24))
    return lax.bitcast_convert_type(w, jnp.int32).reshape(N * 2)


def kernel(coords, grid):
    blocks = jnp.transpose(coords.reshape(N // 128, 128, 3), (0, 2, 1))
    cpl = jnp.pad(blocks, ((0, 0), (0, 1), (0, 0))).reshape(4 * (N // 128), 128)
    pk = _idx_call(cpl)
    tbl = _pack_table(grid)
    return _gat_call(cpl, pk, tbl).reshape(N)


# speedup vs baseline: 6.4660x; 1.3002x over previous
import functools

import jax
import jax.numpy as jnp
from jax import lax
from jax.experimental import pallas as pl
from jax.experimental.pallas import tpu as pltpu
from jax.experimental.pallas import tpu_sc as plsc

N = 2097152
NW = 32
PER_W = N // NW
C = 2048
NCH = PER_W // C
NB = C // 128
ROWS_IN = 4 * NB


def _axis_parts(v):
    ix = v * 128.0 + 127.5
    i0 = (ix + 256.0).astype(jnp.int32) - 256
    w = ix - i0.astype(jnp.float32)
    neg = i0 < 0
    hi = i0 >= 255
    c = jnp.minimum(jnp.maximum(i0, 0), 255)
    a = jnp.where(neg, w, 1.0 - w)
    b = jnp.where(neg | hi, 0.0, w)
    return c, a, b


def _body(cpl, tbl, out, cbuf, axb, bxb, ayb, byb, azb, bzb,
          s8b, idxb, gbuf, obuf, sem0, sem1, osem0, osem1, csem0, csem1):
    wid = lax.axis_index("s") * 2 + lax.axis_index("c")
    sems = [sem0, sem1]
    osems = [osem0, osem1]
    csems = [csem0, csem1]

    def prefetch(j, slot):
        pltpu.async_copy(cpl.at[pl.ds((wid * NCH + j) * ROWS_IN, ROWS_IN)],
                         cbuf.at[slot], csems[slot])

    def stage(j, slot):
        pltpu.make_async_copy(
            cpl.at[pl.ds((wid * NCH + j) * ROWS_IN, ROWS_IN)],
            cbuf.at[slot], csems[slot]).wait()

        def comp_row(r, c2):
            for u in range(8):
                sl = pl.ds(u * 16, 16)
                xc, ax, bx = _axis_parts(cbuf[slot, 4 * r + 0, sl])
                yc, ay, by = _axis_parts(cbuf[slot, 4 * r + 1, sl])
                zc, az, bz = _axis_parts(cbuf[slot, 4 * r + 2, sl])
                idxb[slot, r, sl] = (zc * 16384
                                     + lax.shift_left(
                                         lax.shift_right_logical(yc, 2), 8)
                                     + xc)
                s8b[slot, r, sl] = lax.shift_left(yc & 3, 3)
                axb[slot, r, sl] = ax
                bxb[slot, r, sl] = bx
                ayb[slot, r, sl] = ay
                byb[slot, r, sl] = by
                azb[slot, r, sl] = az
                bzb[slot, r, sl] = bz
            return c2

        lax.fori_loop(0, NB, comp_row, 0, unroll=False)
        for r in range(NB):
            pltpu.async_copy(tbl.at[idxb.at[slot, r]], gbuf.at[slot, r],
                             sems[slot])

    def finish(j, slot):
        for r in range(NB):
            pltpu.make_async_copy(tbl.at[idxb.at[slot, r]],
                                  gbuf.at[slot, r], sems[slot]).wait()

        @pl.when(j >= 2)
        def _():
            pltpu.make_async_copy(
                obuf.at[slot], out.at[pl.ds((wid * NCH + j) * NB, NB)],
                osems[slot]).wait()

        def ext_row(r, c2):
            for u in range(8):
                sl = pl.ds(u * 16, 16)
                w = gbuf[slot, r, sl]
                wsh = lax.shift_right_logical(w, s8b[slot, r, sl])
                ax = axb[slot, r, sl]
                bx = bxb[slot, r, sl]

                def dot(dz, dy):
                    sh = dz * 4 + dy * 2
                    t = lax.shift_right_logical(wsh, sh) if sh else wsh
                    b0 = (t & 1).astype(jnp.float32)
                    b1 = (lax.shift_right_logical(t, 1) & 1).astype(jnp.float32)
                    return ax * b0 + bx * b1

                sz0 = ayb[slot, r, sl] * dot(0, 0) + byb[slot, r, sl] * dot(0, 1)
                sz1 = ayb[slot, r, sl] * dot(1, 0) + byb[slot, r, sl] * dot(1, 1)
                obuf[slot, r, sl] = (azb[slot, r, sl] * sz0
                                     + bzb[slot, r, sl] * sz1)
            return c2

        lax.fori_loop(0, NB, ext_row, 0, unroll=False)
        pltpu.async_copy(obuf.at[slot], out.at[pl.ds((wid * NCH + j) * NB, NB)],
                         osems[slot])

    prefetch(0, 0)
    prefetch(1, 1)
    stage(0, 0)
    prefetch(2, 0)

    def pair(p, carry):
        stage(2 * p + 1, 1)

        @pl.when(p < NCH // 2 - 1)
        def _():
            prefetch(2 * p + 3, 1)

        finish(2 * p, 0)

        @pl.when(p < NCH // 2 - 1)
        def _():
            stage(2 * p + 2, 0)

        @pl.when(p < NCH // 2 - 2)
        def _():
            prefetch(2 * p + 4, 0)

        finish(2 * p + 1, 1)
        return carry

    lax.fori_loop(0, NCH // 2, pair, 0, unroll=False)
    for slot in (0, 1):
        pltpu.make_async_copy(
            obuf.at[slot], out.at[pl.ds(wid * NCH * NB, NB)],
            osems[slot]).wait()


_mesh = plsc.VectorSubcoreMesh(core_axis_name="c", subcore_axis_name="s")

_sc_call = functools.partial(
    pl.kernel,
    mesh=_mesh,
    out_type=jax.ShapeDtypeStruct((N // 128, 128), jnp.float32),
    scratch_types=[
        pltpu.VMEM((2, ROWS_IN, 128), jnp.float32),
        pltpu.VMEM((2, NB, 128), jnp.float32),
        pltpu.VMEM((2, NB, 128), jnp.float32),
        pltpu.VMEM((2, NB, 128), jnp.float32),
        pltpu.VMEM((2, NB, 128), jnp.float32),
        pltpu.VMEM((2, NB, 128), jnp.float32),
        pltpu.VMEM((2, NB, 128), jnp.float32),
        pltpu.VMEM((2, NB, 128), jnp.int32),
        pltpu.VMEM((2, NB, 128), jnp.int32),
        pltpu.VMEM((2, NB, 128), jnp.int32),
        pltpu.VMEM((2, NB, 128), jnp.float32),
        pltpu.SemaphoreType.DMA,
        pltpu.SemaphoreType.DMA,
        pltpu.SemaphoreType.DMA,
        pltpu.SemaphoreType.DMA,
        pltpu.SemaphoreType.DMA,
        pltpu.SemaphoreType.DMA,
    ],
)(_body)


def _pack_table(grid):
    b = lax.optimization_barrier(grid.astype(jnp.uint8))
    px = b | (jnp.pad(b[:, :, 1:], ((0, 0), (0, 0), (0, 1))) << 1)
    pxy = px | (jnp.pad(px[:, 1:, :], ((0, 0), (0, 1), (0, 0))) << 2)
    pxyz = pxy | (jnp.pad(pxy[1:, :, :], ((0, 1), (0, 0), (0, 0))) << 4)
    v = pxyz.reshape(256, 64, 4, 256)
    w = (v[:, :, 0, :].astype(jnp.uint32)
         | (v[:, :, 1, :].astype(jnp.uint32) << 8)
         | (v[:, :, 2, :].astype(jnp.uint32) << 16)
         | (v[:, :, 3, :].astype(jnp.uint32) << 24))
    return lax.bitcast_convert_type(w, jnp.int32).reshape(N * 2)


def kernel(coords, grid):
    blocks = jnp.transpose(coords.reshape(N // 128, 128, 3), (0, 2, 1))
    cpl = jnp.pad(blocks, ((0, 0), (0, 1), (0, 0))).reshape(4 * (N // 128), 128)
    tbl = _pack_table(grid)
    return _sc_call(cpl, tbl).reshape(N)
